# baseline (speedup 1.0000x reference)
import numpy as np
import jax
import jax.numpy as jnp
from jax import lax
from jax.experimental import pallas as pl

_BN_EPS = 1e-5
_H = 22
_W = 22
_N = _H * _W
_NP = 512
_NKP = 16
_NPAIR = _NKP * _NKP
_DIRS = (1, 21, 22, 23)
_NEDGE = len(_DIRS) * _NP
_XROWS = _NEDGE + _NPAIR
_BF = jnp.bfloat16
_F32 = jnp.float32


_BS = 4


def _fused(s_ref, xc_ref, kp_ref, kp0_ref, kp1_ref, w1_ref, b1_ref, g_ref,
           be_ref, mu_ref, var_ref, w2t_ref, w2p_ref, b2_ref, gc1_ref,
           gc2t_ref, out_ref):
  for _s in range(_BS):
    f_cm = jnp.concatenate([s_ref[_s], xc_ref[_s]],
                           axis=0).astype(_F32)
    f_cm = jnp.pad(f_cm, ((0, 0), (0, _NP - _N)))
    f = f_cm.T

    w1b = w1_ref[...].astype(_BF)
    scale = g_ref[...] * lax.rsqrt(var_ref[...] + _BN_EPS)
    shift = be_ref[...] - mu_ref[...] * scale
    b1s = b1_ref[...] * scale + shift
    b2 = b2_ref[0, 0]

    def _phi_edges(diff_bf):
        h = jnp.dot(diff_bf, w1b, preferred_element_type=_F32) * scale + b1s
        h = jnp.maximum(h, 0.0)
        ef = jnp.dot(h, w2p_ref[...], preferred_element_type=_F32)
        return jax.nn.sigmoid(lax.slice(ef, (0, 0), (diff_bf.shape[0], 1)) + b2)

    base = lax.slice(f, (0, 0), (_N, 512))
    e_ds = []
    for d in _DIRS:
        sh = lax.slice(f, (d, 0), (d + _N, 512))
        dd = jnp.abs(base - sh).astype(_BF)
        e_ds.append(_phi_edges(dd))

    kpv = kp_ref[_s]
    rows16 = lax.broadcasted_iota(jnp.int32, (_NP, _NKP), 0)
    PT = (rows16 == jnp.broadcast_to(kpv, (_NP, _NKP))).astype(_BF)
    fk = lax.dot_general(PT, f.astype(_BF), (((0,), (0,)), ((), ())),
                         preferred_element_type=_F32)
    pi = lax.broadcasted_iota(jnp.int32, (_NPAIR, _NKP), 0)
    ci = lax.broadcasted_iota(jnp.int32, (_NPAIR, _NKP), 1)
    Rii = ((pi // _NKP) == ci).astype(_BF)
    Rjj = ((pi % _NKP) == ci).astype(_BF)
    fkb = fk.astype(_BF)
    f1k = jnp.dot(Rii, fkb, preferred_element_type=_F32)
    f2k = jnp.dot(Rjj, fkb, preferred_element_type=_F32)
    dk = jnp.abs(f1k - f2k).astype(_BF)
    hk = jnp.dot(dk, w1b, preferred_element_type=_F32) * scale + b1s
    hk = jnp.maximum(hk, 0.0)
    e_k = jax.nn.sigmoid(
        lax.dot_general(w2t_ref[...], hk, (((1,), (1,)), ((), ())),
                        preferred_element_type=_F32) + b2)
    kp0v = kp0_ref[_s]
    kp1v = kp1_ref[_s]
    e_k = jnp.where(kp0v == kp1v, 0.0, e_k)

    R = lax.broadcasted_iota(jnp.int32, (_NP, _NP), 0)
    Cc = lax.broadcasted_iota(jnp.int32, (_NP, _NP), 1)
    u = lax.broadcasted_iota(jnp.int32, (_NP, 1), 0)
    xc = u % _W
    valid = {
        1: (xc < _W - 1) & (u < _N),
        21: (xc >= 1) & (u < _N - _W + 1),
        22: u < _N - _W,
        23: (xc < _W - 1) & (u < _N - _W - 1),
    }
    dcol = Cc - R
    U = jnp.zeros((_NP, _NP), _F32)
    for k, d in enumerate(_DIRS):
        ed = jnp.pad(e_ds[k], ((0, _NP - _N), (0, 0)))
        ed = jnp.where(valid[d], ed, 0.0)
        U = jnp.where(dcol == d, jnp.broadcast_to(ed, (_NP, _NP)), U)
    An = U + U.T

    rows256 = lax.broadcasted_iota(jnp.int32, (_NP, _NPAIR), 0)
    PT0 = (rows256 == jnp.broadcast_to(kp0v, (_NP, _NPAIR))).astype(_BF)
    PT1 = (rows256 == jnp.broadcast_to(kp1v, (_NP, _NPAIR))).astype(_BF)
    Sw = lax.dot_general(PT0 * e_k.astype(_BF), PT1, (((1,), (1,)), ((), ())),
                         preferred_element_type=_F32)
    Cnt = lax.dot_general(PT0, PT1, (((1,), (1,)), ((), ())),
                          preferred_element_type=_F32)
    A = jnp.where(Cnt > 0.5, Sw / jnp.maximum(Cnt, 1.0), An)

    y1t = lax.dot_general(gc1_ref[...], f, (((0,), (1,)), ((), ())),
                          preferred_element_type=_F32)
    x1t = jnp.dot(y1t, A, preferred_element_type=_F32)
    x1t = jnp.where(x1t >= 0, x1t, 0.2 * x1t)
    z = jnp.dot(gc2t_ref[...], x1t, preferred_element_type=_F32)
    m = jax.nn.sigmoid(jnp.dot(z, A, preferred_element_type=_F32))
    out_ref[_s] = m


def kernel(search_feature, xcorr_map, saliency, key_coords, graph_size,
           phi_w1, phi_b1, bn_gamma, bn_beta, bn_mean, bn_var,
           phi_w2, phi_b2, gc1_w, gc2_w):
    B = search_feature.shape[0]
    s_r = search_feature.reshape(B, 256, _N).astype(_BF)
    x_r = xcorr_map.reshape(B, 256, _N).astype(_BF)

    kpf = (key_coords[:, :, 0] * _W + key_coords[:, :, 1]).astype(jnp.int32)
    kp0 = jnp.repeat(kpf, _NKP, axis=1)[:, None, :]
    kp1 = jnp.tile(kpf, (1, _NKP))[:, None, :]
    kp3 = kpf[:, None, :]

    b1r = phi_b1.reshape(1, 256)
    gr = bn_gamma.reshape(1, 256)
    ber = bn_beta.reshape(1, 256)
    mur = bn_mean.reshape(1, 256)
    varr = bn_var.reshape(1, 256)
    w2t = phi_w2.reshape(1, 256)
    w2p = jnp.pad(phi_w2, ((0, 0), (0, 127)))
    b2r = phi_b2.reshape(1, 1)
    gc2t = gc2_w.reshape(1, 128)

    out = pl.pallas_call(
        _fused,
        grid=(B // _BS,),
        in_specs=[
            pl.BlockSpec((_BS, 256, _N), lambda b: (b, 0, 0)),
            pl.BlockSpec((_BS, 256, _N), lambda b: (b, 0, 0)),
            pl.BlockSpec((_BS, 1, _NKP), lambda b: (b, 0, 0)),
            pl.BlockSpec((_BS, 1, _NPAIR), lambda b: (b, 0, 0)),
            pl.BlockSpec((_BS, 1, _NPAIR), lambda b: (b, 0, 0)),
            pl.BlockSpec((512, 256), lambda b: (0, 0)),
            pl.BlockSpec((1, 256), lambda b: (0, 0)),
            pl.BlockSpec((1, 256), lambda b: (0, 0)),
            pl.BlockSpec((1, 256), lambda b: (0, 0)),
            pl.BlockSpec((1, 256), lambda b: (0, 0)),
            pl.BlockSpec((1, 256), lambda b: (0, 0)),
            pl.BlockSpec((1, 256), lambda b: (0, 0)),
            pl.BlockSpec((256, 128), lambda b: (0, 0)),
            pl.BlockSpec((1, 1), lambda b: (0, 0)),
            pl.BlockSpec((512, 128), lambda b: (0, 0)),
            pl.BlockSpec((1, 128), lambda b: (0, 0)),
        ],
        out_specs=pl.BlockSpec((_BS, 1, _NP), lambda b: (b, 0, 0)),
        out_shape=jax.ShapeDtypeStruct((B, 1, _NP), jnp.float32),
    )(s_r, x_r, kp3, kp0, kp1, phi_w1, b1r, gr, ber, mur, varr, w2t, w2p, b2r,
      gc1_w, gc2t)

    return out[:, 0, :_N].reshape(B, _H, _W)[:, None, :, :]


# speedup vs baseline: 1.0407x; 1.0407x over previous
import numpy as np
import jax
import jax.numpy as jnp
from jax import lax
from jax.experimental import pallas as pl

_BN_EPS = 1e-5
_H = 22
_W = 22
_N = _H * _W
_NP = 512
_NKP = 16
_NPAIR = _NKP * _NKP
_DIRS = (1, 21, 22, 23)
_BF = jnp.bfloat16
_F32 = jnp.float32
_BS = 4


def _fused(s_ref, xc_ref, kp_ref, kp0_ref, kp1_ref, w1_ref, b1_ref, g_ref,
           be_ref, mu_ref, var_ref, w2t_ref, w2p_ref, b2_ref, gc1_ref,
           gc2t_ref, out_ref):
  for _s in range(_BS):
    f_cm = jnp.concatenate([s_ref[_s], xc_ref[_s]], axis=0)
    f_cm = jnp.pad(f_cm, ((0, 0), (0, _NP - _N)))
    f = f_cm.T

    w1b = w1_ref[...].astype(_BF)
    scale = g_ref[...] * lax.rsqrt(var_ref[...] + _BN_EPS)
    shift = be_ref[...] - mu_ref[...] * scale
    b1s = b1_ref[...] * scale + shift
    b2 = b2_ref[0, 0]

    def _phi_edges(diff_bf):
        h = jnp.dot(diff_bf, w1b, preferred_element_type=_F32) * scale + b1s
        h = jnp.maximum(h, 0.0)
        ef = jnp.dot(h, w2p_ref[...], preferred_element_type=_F32)
        return jax.nn.sigmoid(lax.slice(ef, (0, 0), (diff_bf.shape[0], 1)) + b2)

    base = lax.slice(f, (0, 0), (_N, 512))
    e_ds = []
    for d in _DIRS:
        sh = lax.slice(f, (d, 0), (d + _N, 512))
        dd = jnp.abs(base - sh).astype(_BF)
        e_ds.append(_phi_edges(dd))

    kpv = kp_ref[_s]
    rows16 = lax.broadcasted_iota(jnp.int32, (_NP, _NKP), 0)
    PT = (rows16 == jnp.broadcast_to(kpv, (_NP, _NKP))).astype(_BF)
    fk = lax.dot_general(PT, f.astype(_BF), (((0,), (0,)), ((), ())),
                         preferred_element_type=_F32)
    pi = lax.broadcasted_iota(jnp.int32, (_NPAIR, _NKP), 0)
    ci = lax.broadcasted_iota(jnp.int32, (_NPAIR, _NKP), 1)
    Rii = ((pi // _NKP) == ci).astype(_BF)
    Rjj = ((pi % _NKP) == ci).astype(_BF)
    fkb = fk.astype(_BF)
    f1k = jnp.dot(Rii, fkb, preferred_element_type=_F32)
    f2k = jnp.dot(Rjj, fkb, preferred_element_type=_F32)
    dk = jnp.abs(f1k - f2k).astype(_BF)
    hk = jnp.dot(dk, w1b, preferred_element_type=_F32) * scale + b1s
    hk = jnp.maximum(hk, 0.0)
    e_k = jax.nn.sigmoid(
        lax.dot_general(w2t_ref[...], hk, (((1,), (1,)), ((), ())),
                        preferred_element_type=_F32) + b2)
    kp0v = kp0_ref[_s]
    kp1v = kp1_ref[_s]
    e_k = jnp.where(kp0v == kp1v, 0.0, e_k)

    R = lax.broadcasted_iota(jnp.int32, (_NP, _NP), 0)
    Cc = lax.broadcasted_iota(jnp.int32, (_NP, _NP), 1)
    u = lax.broadcasted_iota(jnp.int32, (_NP, 1), 0)
    xc = u % _W
    valid = {
        1: (xc < _W - 1) & (u < _N),
        21: (xc >= 1) & (u < _N - _W + 1),
        22: u < _N - _W,
        23: (xc < _W - 1) & (u < _N - _W - 1),
    }
    dcol = Cc - R
    U = jnp.zeros((_NP, _NP), _F32)
    for k, d in enumerate(_DIRS):
        ed = jnp.pad(e_ds[k], ((0, _NP - _N), (0, 0)))
        ed = jnp.where(valid[d], ed, 0.0)
        U = jnp.where(dcol == d, jnp.broadcast_to(ed, (_NP, _NP)), U)
    An = U + U.T

    rows256 = lax.broadcasted_iota(jnp.int32, (_NP, _NPAIR), 0)
    PT0 = (rows256 == jnp.broadcast_to(kp0v, (_NP, _NPAIR))).astype(_BF)
    PT1 = (rows256 == jnp.broadcast_to(kp1v, (_NP, _NPAIR))).astype(_BF)
    Sw = lax.dot_general(PT0 * e_k.astype(_BF), PT1, (((1,), (1,)), ((), ())),
                         preferred_element_type=_F32)
    Cnt = lax.dot_general(PT0, PT1, (((1,), (1,)), ((), ())),
                          preferred_element_type=_F32)
    A = jnp.where(Cnt > 0.5, Sw / jnp.maximum(Cnt, 1.0), An)

    y1t = lax.dot_general(gc1_ref[...], f, (((0,), (1,)), ((), ())),
                          preferred_element_type=_F32)
    x1t = jnp.dot(y1t, A, preferred_element_type=_F32)
    x1t = jnp.where(x1t >= 0, x1t, 0.2 * x1t)
    z = jnp.dot(gc2t_ref[...], x1t, preferred_element_type=_F32)
    m = jax.nn.sigmoid(jnp.dot(z, A, preferred_element_type=_F32))
    out_ref[_s] = m


def kernel(search_feature, xcorr_map, saliency, key_coords, graph_size,
           phi_w1, phi_b1, bn_gamma, bn_beta, bn_mean, bn_var,
           phi_w2, phi_b2, gc1_w, gc2_w):
    B = search_feature.shape[0]
    s_r = search_feature.reshape(B, 256, _N)
    x_r = xcorr_map.reshape(B, 256, _N)

    kpf = (key_coords[:, :, 0] * _W + key_coords[:, :, 1]).astype(jnp.int32)
    kp0 = jnp.repeat(kpf, _NKP, axis=1)[:, None, :]
    kp1 = jnp.tile(kpf, (1, _NKP))[:, None, :]
    kp3 = kpf[:, None, :]

    b1r = phi_b1.reshape(1, 256)
    gr = bn_gamma.reshape(1, 256)
    ber = bn_beta.reshape(1, 256)
    mur = bn_mean.reshape(1, 256)
    varr = bn_var.reshape(1, 256)
    w2t = phi_w2.reshape(1, 256)
    w2p = jnp.pad(phi_w2, ((0, 0), (0, 127)))
    b2r = phi_b2.reshape(1, 1)
    gc2t = gc2_w.reshape(1, 128)

    out = pl.pallas_call(
        _fused,
        grid=(B // _BS,),
        in_specs=[
            pl.BlockSpec((_BS, 256, _N), lambda b: (b, 0, 0)),
            pl.BlockSpec((_BS, 256, _N), lambda b: (b, 0, 0)),
            pl.BlockSpec((_BS, 1, _NKP), lambda b: (b, 0, 0)),
            pl.BlockSpec((_BS, 1, _NPAIR), lambda b: (b, 0, 0)),
            pl.BlockSpec((_BS, 1, _NPAIR), lambda b: (b, 0, 0)),
            pl.BlockSpec((512, 256), lambda b: (0, 0)),
            pl.BlockSpec((1, 256), lambda b: (0, 0)),
            pl.BlockSpec((1, 256), lambda b: (0, 0)),
            pl.BlockSpec((1, 256), lambda b: (0, 0)),
            pl.BlockSpec((1, 256), lambda b: (0, 0)),
            pl.BlockSpec((1, 256), lambda b: (0, 0)),
            pl.BlockSpec((1, 256), lambda b: (0, 0)),
            pl.BlockSpec((256, 128), lambda b: (0, 0)),
            pl.BlockSpec((1, 1), lambda b: (0, 0)),
            pl.BlockSpec((512, 128), lambda b: (0, 0)),
            pl.BlockSpec((1, 128), lambda b: (0, 0)),
        ],
        out_specs=pl.BlockSpec((_BS, 1, _NP), lambda b: (b, 0, 0)),
        out_shape=jax.ShapeDtypeStruct((B, 1, _NP), jnp.float32),
    )(s_r, x_r, kp3, kp0, kp1, phi_w1, b1r, gr, ber, mur, varr, w2t, w2p, b2r,
      gc1_w, gc2t)

    return out[:, 0, :_N].reshape(B, _H, _W)[:, None, :, :]
